# baseline (speedup 1.0000x reference)
import functools

import jax
import jax.numpy as jnp
from jax import lax
from jax.experimental import pallas as pl
from jax.experimental.pallas import tpu as pltpu
from jax.experimental.pallas import tpu_sc as plsc

N_NODES = 10000
N_EDGES = 320000
D_HID = 128
D_EDGE = 16
N_GRAPHS = 64

SC_CORES = 2
SC_SUBCORES = 16
LANES = 16
N_WORKERS = SC_CORES * SC_SUBCORES
EDGES_PER_WORKER = N_EDGES // N_WORKERS
CHUNK = 40
N_CHUNKS = EDGES_PER_WORKER // CHUNK
STRIPE_ROWS = 40
N_STRIPES = N_NODES // STRIPE_ROWS

NODE_BLK = 1000
N_NODE_BLKS = N_NODES // NODE_BLK
EDGE_BLK = 2000
N_EDGE_BLKS = N_EDGES // EDGE_BLK

_LOG1P = (
    5.629329952183681e-07, 0.9999574661581281, -0.49920638240556336,
    0.3269723524228364, -0.22283471747823236, 0.13076335879271853,
    -0.05262395515996885, 0.010118901693937057,
)


def _node_proj_body(x_ref, wpre_ref, bpre_ref, wq_ref, wr_ref,
                    h_ref, q_ref, r_ref):
    h = jnp.dot(x_ref[:], wpre_ref[:], preferred_element_type=jnp.float32, precision=lax.Precision.HIGHEST)
    h = jnp.maximum(h + bpre_ref[:], 0.0)
    h_ref[:] = h
    q_ref[:] = jnp.dot(h, wq_ref[:], preferred_element_type=jnp.float32, precision=lax.Precision.HIGHEST)
    r_ref[:] = jnp.dot(h, wr_ref[:], preferred_element_type=jnp.float32, precision=lax.Precision.HIGHEST)


def _mid_proj_body(h_ref, a0_ref, a1_ref, wq_ref, wr_ref,
                   hn_ref, q_ref, r_ref):
    h = jnp.maximum(h_ref[:] + a0_ref[:] + a1_ref[:], 0.0)
    hn_ref[:] = h
    q_ref[:] = jnp.dot(h, wq_ref[:], preferred_element_type=jnp.float32, precision=lax.Precision.HIGHEST)
    r_ref[:] = jnp.dot(h, wr_ref[:], preferred_element_type=jnp.float32, precision=lax.Precision.HIGHEST)


def _edge_proj_body(ea_ref, w1_ref, b1_ref, w2_ref, b2_ref, t1_ref, t2_ref):
    ea = ea_ref[:]
    t1_ref[:] = jnp.dot(ea, w1_ref[:], preferred_element_type=jnp.float32, precision=lax.Precision.HIGHEST) + b1_ref[:]
    t2_ref[:] = jnp.dot(ea, w2_ref[:], preferred_element_type=jnp.float32, precision=lax.Precision.HIGHEST) + b2_ref[:]


def _pool_head_body(h_ref, a0_ref, a1_ref, batch_ref,
                    w1_ref, b1_ref, w2_ref, b2_ref, wo_ref, bo_ref,
                    out_ref, sum_acc, cnt_acc):
    i = pl.program_id(0)

    @pl.when(i == 0)
    def _():
        sum_acc[:] = jnp.zeros_like(sum_acc)
        cnt_acc[:] = jnp.zeros_like(cnt_acc)

    h2 = jnp.maximum(h_ref[:] + a0_ref[:] + a1_ref[:], 0.0)
    b = batch_ref[0, 0, :]
    gids = lax.broadcasted_iota(jnp.int32, (N_GRAPHS, NODE_BLK), 0)
    onehot_t = (b[None, :] == gids).astype(jnp.float32)
    sum_acc[:] += jnp.dot(onehot_t, h2, preferred_element_type=jnp.float32, precision=lax.Precision.HIGHEST)
    cnt = jnp.sum(onehot_t, axis=1, keepdims=True)
    cnt_acc[:] += jnp.broadcast_to(cnt, cnt_acc.shape)

    @pl.when(i == N_NODE_BLKS - 1)
    def _():
        pooled = sum_acc[:] / jnp.maximum(cnt_acc[:], 1.0)
        o = jnp.dot(pooled, w1_ref[:], preferred_element_type=jnp.float32, precision=lax.Precision.HIGHEST)
        o = jnp.maximum(o + b1_ref[:], 0.0)
        o = jnp.dot(o, w2_ref[:], preferred_element_type=jnp.float32, precision=lax.Precision.HIGHEST)
        o = jnp.maximum(o + b2_ref[:], 0.0)
        o = jnp.dot(o, wo_ref[:], preferred_element_type=jnp.float32, precision=lax.Precision.HIGHEST)
        out_ref[:] = o + bo_ref[:]


def _full(shape):
    return pl.BlockSpec(shape, lambda i: (0,) * len(shape))


def _rows(blk, width):
    return pl.BlockSpec((blk, width), lambda i: (i, 0))


def _sc_layer_body(q_hbm, r_hbm, t_hbm, src_hbm, dst_hbm, zero_hbm, out_hbm,
                   srcv, dstv, qbuf, rbuf, tbuf, mbuf, agg, sem_q, sem_r):
    cid = lax.axis_index("c")
    sid = lax.axis_index("s")
    wid = cid * SC_SUBCORES + sid

    n_stripes = (N_STRIPES - 1 - sid) // SC_SUBCORES + 1

    def zero_body(k, c):
        off = pl.multiple_of((k * SC_SUBCORES + sid) * STRIPE_ROWS, 8)
        pltpu.sync_copy(zero_hbm.at[pl.ds(off, STRIPE_ROWS)],
                        agg.at[pl.ds(off, STRIPE_ROWS)])
        return c

    lax.fori_loop(0, n_stripes, zero_body, 0)
    plsc.subcore_barrier()

    def chunk_body(g, carry):
        base = pl.multiple_of(wid * EDGES_PER_WORKER + g * CHUNK, CHUNK)
        pltpu.sync_copy(src_hbm.at[pl.ds(base, CHUNK)], srcv)
        pltpu.sync_copy(dst_hbm.at[pl.ds(base, CHUNK)], dstv)
        cq = pltpu.async_copy(q_hbm.at[dstv], qbuf, sem_q)
        cr = pltpu.async_copy(r_hbm.at[srcv], rbuf, sem_r)
        pltpu.sync_copy(t_hbm.at[pl.ds(base, CHUNK)], tbuf)
        cq.wait()
        cr.wait()

        def edge_body(e, ecarry):
            for j in range(D_HID // LANES):
                lo = j * LANES
                hi = D_HID + lo
                af = (qbuf[e, pl.ds(lo, LANES)] + rbuf[e, pl.ds(lo, LANES)]
                      + tbuf[e, pl.ds(lo, LANES)])
                av = (qbuf[e, pl.ds(hi, LANES)] + rbuf[e, pl.ds(hi, LANES)]
                      + tbuf[e, pl.ds(hi, LANES)])
                sig = 1.0 / (1.0 + jnp.exp(-af))
                t = jnp.exp(-jnp.abs(av))
                p = jnp.full((LANES,), _LOG1P[7], jnp.float32)
                for c in _LOG1P[6::-1]:
                    p = p * t + c
                sp = jnp.maximum(av, 0.0) + p
                mbuf[e, pl.ds(lo, LANES)] = sig * sp
            return ecarry

        lax.fori_loop(0, CHUNK, edge_body, 0)
        pltpu.sync_copy(mbuf, agg.at[dstv], add=True)
        return carry

    lax.fori_loop(0, N_CHUNKS, chunk_body, 0)
    plsc.subcore_barrier()

    def out_body(k, c):
        off = pl.multiple_of((k * SC_SUBCORES + sid) * STRIPE_ROWS, 8)
        pltpu.sync_copy(agg.at[pl.ds(off, STRIPE_ROWS)],
                        out_hbm.at[cid, pl.ds(off, STRIPE_ROWS)])
        return c

    lax.fori_loop(0, n_stripes, out_body, 0)


def _make_sc_layer():
    return pl.kernel(
        _sc_layer_body,
        out_type=jax.ShapeDtypeStruct((SC_CORES, N_NODES, D_HID), jnp.float32),
        mesh=plsc.VectorSubcoreMesh(
            core_axis_name="c", subcore_axis_name="s",
            num_cores=SC_CORES, num_subcores=SC_SUBCORES),
        scratch_types=[
            pltpu.VMEM((CHUNK,), jnp.int32),
            pltpu.VMEM((CHUNK,), jnp.int32),
            pltpu.VMEM((CHUNK, 2 * D_HID), jnp.float32),
            pltpu.VMEM((CHUNK, 2 * D_HID), jnp.float32),
            pltpu.VMEM((CHUNK, 2 * D_HID), jnp.float32),
            pltpu.VMEM((CHUNK, D_HID), jnp.float32),
            pltpu.VMEM_SHARED((N_NODES, D_HID), jnp.float32),
            pltpu.SemaphoreType.DMA,
            pltpu.SemaphoreType.DMA,
        ],
    )


def kernel(x, edge_index, edge_attr, batch, Wpre, bpre, Wf1, bf1, Ws1, bs1,
           Wf2, bf2, Ws2, bs2, W1, b1, W2, b2, Wout, bout):
    src = edge_index[0]
    dst = edge_index[1]

    wq1 = jnp.concatenate([Wf1[:D_HID], Ws1[:D_HID]], axis=1)
    wr1 = jnp.concatenate([Wf1[D_HID:2 * D_HID], Ws1[D_HID:2 * D_HID]], axis=1)
    wq2 = jnp.concatenate([Wf2[:D_HID], Ws2[:D_HID]], axis=1)
    wr2 = jnp.concatenate([Wf2[D_HID:2 * D_HID], Ws2[D_HID:2 * D_HID]], axis=1)
    we1 = jnp.concatenate([Wf1[2 * D_HID:], Ws1[2 * D_HID:]], axis=1)
    we2 = jnp.concatenate([Wf2[2 * D_HID:], Ws2[2 * D_HID:]], axis=1)
    be1 = jnp.concatenate([bf1, bs1]).reshape(1, 2 * D_HID)
    be2 = jnp.concatenate([bf2, bs2]).reshape(1, 2 * D_HID)

    h0, q1, r1 = pl.pallas_call(
        _node_proj_body,
        grid=(N_NODE_BLKS,),
        in_specs=[
            _rows(NODE_BLK, D_HID),
            _full((D_HID, D_HID)),
            _full((1, D_HID)),
            _full((D_HID, 2 * D_HID)),
            _full((D_HID, 2 * D_HID)),
        ],
        out_specs=[
            _rows(NODE_BLK, D_HID),
            _rows(NODE_BLK, 2 * D_HID),
            _rows(NODE_BLK, 2 * D_HID),
        ],
        out_shape=[
            jax.ShapeDtypeStruct((N_NODES, D_HID), jnp.float32),
            jax.ShapeDtypeStruct((N_NODES, 2 * D_HID), jnp.float32),
            jax.ShapeDtypeStruct((N_NODES, 2 * D_HID), jnp.float32),
        ],
    )(x, Wpre, bpre.reshape(1, D_HID), wq1, wr1)

    t1, t2 = pl.pallas_call(
        _edge_proj_body,
        grid=(N_EDGE_BLKS,),
        in_specs=[
            _rows(EDGE_BLK, D_EDGE),
            _full((D_EDGE, 2 * D_HID)),
            _full((1, 2 * D_HID)),
            _full((D_EDGE, 2 * D_HID)),
            _full((1, 2 * D_HID)),
        ],
        out_specs=[
            _rows(EDGE_BLK, 2 * D_HID),
            _rows(EDGE_BLK, 2 * D_HID),
        ],
        out_shape=[
            jax.ShapeDtypeStruct((N_EDGES, 2 * D_HID), jnp.float32),
            jax.ShapeDtypeStruct((N_EDGES, 2 * D_HID), jnp.float32),
        ],
    )(edge_attr, we1, be1, we2, be2)

    zeros = jnp.zeros((N_NODES, D_HID), jnp.float32)

    sc_layer = _make_sc_layer()
    agg1 = sc_layer(q1, r1, t1, src, dst, zeros)

    h1, q2, r2 = pl.pallas_call(
        _mid_proj_body,
        grid=(N_NODE_BLKS,),
        in_specs=[
            _rows(NODE_BLK, D_HID),
            _rows(NODE_BLK, D_HID),
            _rows(NODE_BLK, D_HID),
            _full((D_HID, 2 * D_HID)),
            _full((D_HID, 2 * D_HID)),
        ],
        out_specs=[
            _rows(NODE_BLK, D_HID),
            _rows(NODE_BLK, 2 * D_HID),
            _rows(NODE_BLK, 2 * D_HID),
        ],
        out_shape=[
            jax.ShapeDtypeStruct((N_NODES, D_HID), jnp.float32),
            jax.ShapeDtypeStruct((N_NODES, 2 * D_HID), jnp.float32),
            jax.ShapeDtypeStruct((N_NODES, 2 * D_HID), jnp.float32),
        ],
    )(h0, agg1[0], agg1[1], wq2, wr2)

    agg2 = sc_layer(q2, r2, t2, src, dst, zeros)

    out = pl.pallas_call(
        _pool_head_body,
        grid=(N_NODE_BLKS,),
        in_specs=[
            _rows(NODE_BLK, D_HID),
            _rows(NODE_BLK, D_HID),
            _rows(NODE_BLK, D_HID),
            pl.BlockSpec((1, 1, NODE_BLK), lambda i: (i, 0, 0)),
            _full((D_HID, D_HID)),
            _full((1, D_HID)),
            _full((D_HID, D_HID)),
            _full((1, D_HID)),
            _full((D_HID, 3)),
            _full((1, 3)),
        ],
        out_specs=pl.BlockSpec((N_GRAPHS, 3), lambda i: (0, 0)),
        out_shape=jax.ShapeDtypeStruct((N_GRAPHS, 3), jnp.float32),
        scratch_shapes=[
            pltpu.VMEM((N_GRAPHS, D_HID), jnp.float32),
            pltpu.VMEM((N_GRAPHS, D_HID), jnp.float32),
        ],
    )(h1, agg2[0], agg2[1], batch.reshape(N_NODE_BLKS, 1, NODE_BLK),
      W1, b1.reshape(1, D_HID), W2, b2.reshape(1, D_HID),
      Wout, bout.reshape(1, 3))

    return out


# speedup vs baseline: 2.4686x; 2.4686x over previous
import jax
import jax.numpy as jnp
from jax import lax
from jax.experimental import pallas as pl
from jax.experimental.pallas import tpu as pltpu
from jax.experimental.pallas import tpu_sc as plsc

N_NODES = 10000
N_EDGES = 320000
D_HID = 128
D_EDGE = 16
N_GRAPHS = 64

SC_CORES = 2
SC_SUBCORES = 16
LANES = 16
N_GROUPS = D_HID // LANES
N_WORKERS = SC_CORES * SC_SUBCORES
EDGES_PER_WORKER = N_EDGES // N_WORKERS
CHUNK = 24
EPW_PAD = 10032
N_CHUNKS = EPW_PAD // CHUNK
E_PAD = N_WORKERS * EPW_PAD
N_CHUNK_ROWS = E_PAD // CHUNK
N_ACC = N_NODES + 8
GARBAGE_ROW = N_NODES
STRIPE_ROWS = 40
N_STRIPES = N_NODES // STRIPE_ROWS

NODE_BLK = 1000
N_NODE_BLKS = N_NODES // NODE_BLK
EDGE_BLK = 2112
N_EDGE_BLKS = E_PAD // EDGE_BLK

_LOG1P = (
    2.2132784001038797e-05, 0.9990102089269602, -0.48915578201144777,
    0.28330238362042115, -0.13011793028847676, 0.030102247599677626,
)


def _node_proj_body(x_ref, wpre_ref, bpre_ref, wq_ref, wr_ref,
                    h_ref, q_ref, r_ref):
    h = jnp.dot(x_ref[:], wpre_ref[:], preferred_element_type=jnp.float32,
                precision=lax.Precision.HIGHEST)
    h = jnp.maximum(h + bpre_ref[:], 0.0)
    h_ref[:] = h
    q_ref[:] = jnp.dot(h, wq_ref[:], preferred_element_type=jnp.float32,
                       precision=lax.Precision.HIGHEST)
    r_ref[:] = jnp.dot(h, wr_ref[:], preferred_element_type=jnp.float32,
                       precision=lax.Precision.HIGHEST)


def _mid_proj_body(h_ref, a0_ref, a1_ref, wq_ref, wr_ref,
                   hn_ref, q_ref, r_ref):
    h = jnp.maximum(h_ref[:] + a0_ref[:] + a1_ref[:], 0.0)
    hn_ref[:] = h
    q_ref[:] = jnp.dot(h, wq_ref[:], preferred_element_type=jnp.float32,
                       precision=lax.Precision.HIGHEST)
    r_ref[:] = jnp.dot(h, wr_ref[:], preferred_element_type=jnp.float32,
                       precision=lax.Precision.HIGHEST)


def _edge_proj_body(ea_ref, w1_ref, b1_ref, w2_ref, b2_ref, t1_ref, t2_ref):
    ea = ea_ref[:]
    t1_ref[:] = jnp.dot(ea, w1_ref[:], preferred_element_type=jnp.float32,
                        precision=lax.Precision.HIGHEST) + b1_ref[:]
    t2_ref[:] = jnp.dot(ea, w2_ref[:], preferred_element_type=jnp.float32,
                        precision=lax.Precision.HIGHEST) + b2_ref[:]


def _pool_head_body(h_ref, a0_ref, a1_ref, batch_ref,
                    w1_ref, b1_ref, w2_ref, b2_ref, wo_ref, bo_ref,
                    out_ref, sum_acc, cnt_acc):
    i = pl.program_id(0)

    @pl.when(i == 0)
    def _():
        sum_acc[:] = jnp.zeros_like(sum_acc)
        cnt_acc[:] = jnp.zeros_like(cnt_acc)

    h2 = jnp.maximum(h_ref[:] + a0_ref[:] + a1_ref[:], 0.0)
    b = batch_ref[0, 0, :]
    gids = lax.broadcasted_iota(jnp.int32, (N_GRAPHS, NODE_BLK), 0)
    onehot_t = (b[None, :] == gids).astype(jnp.float32)
    sum_acc[:] += jnp.dot(onehot_t, h2, preferred_element_type=jnp.float32,
                          precision=lax.Precision.HIGHEST)
    cnt = jnp.sum(onehot_t, axis=1, keepdims=True)
    cnt_acc[:] += jnp.broadcast_to(cnt, cnt_acc.shape)

    @pl.when(i == N_NODE_BLKS - 1)
    def _():
        pooled = sum_acc[:] / jnp.maximum(cnt_acc[:], 1.0)
        o = jnp.dot(pooled, w1_ref[:], preferred_element_type=jnp.float32,
                    precision=lax.Precision.HIGHEST)
        o = jnp.maximum(o + b1_ref[:], 0.0)
        o = jnp.dot(o, w2_ref[:], preferred_element_type=jnp.float32,
                    precision=lax.Precision.HIGHEST)
        o = jnp.maximum(o + b2_ref[:], 0.0)
        o = jnp.dot(o, wo_ref[:], preferred_element_type=jnp.float32,
                    precision=lax.Precision.HIGHEST)
        out_ref[:] = o + bo_ref[:]


def _full(shape):
    return pl.BlockSpec(shape, lambda i: (0,) * len(shape))


def _rows(blk, width):
    return pl.BlockSpec((blk, width), lambda i: (i, 0))


def _gates(af, av):
    sig = 1.0 / (1.0 + jnp.exp(-af))
    t = jnp.exp(-jnp.abs(av))
    p = jnp.full((LANES,), _LOG1P[5], jnp.float32)
    for c in _LOG1P[4::-1]:
        p = p * t + c
    sp = jnp.maximum(av, 0.0) + p
    return sig * sp


def _sc_layer_body(q_hbm, r_hbm, t_hbm, ei_hbm, zero_hbm, out_hbm,
                   ib0, ib1, qb0, qb1, rb0, rb1, tb0, tb1, mb0, mb1, agg,
                   sQ0, sQ1, sR0, sR1, sT0, sT1, sI0, sI1):
    cid = lax.axis_index("c")
    sid = lax.axis_index("s")
    wid = cid * SC_SUBCORES + sid
    row0 = wid * N_CHUNKS

    ib = (ib0, ib1)
    qb = (qb0, qb1)
    rb = (rb0, rb1)
    tb = (tb0, tb1)
    mb = (mb0, mb1)
    sQ = (sQ0, sQ1)
    sR = (sR0, sR1)
    sT = (sT0, sT1)
    sI = (sI0, sI1)

    n_stripes = (N_STRIPES - 1 - sid) // SC_SUBCORES + 1

    def zero_body(k, c):
        off = pl.multiple_of((k * SC_SUBCORES + sid) * STRIPE_ROWS, 8)
        pltpu.sync_copy(zero_hbm.at[pl.ds(off, STRIPE_ROWS)],
                        agg.at[pl.ds(off, STRIPE_ROWS)])
        return c

    lax.fori_loop(0, n_stripes, zero_body, 0)
    plsc.subcore_barrier()

    def issue_gathers(g, s):
        tbase = pl.multiple_of((row0 + g) * CHUNK, 8)
        pltpu.async_copy(q_hbm.at[ib[s].at[1]], qb[s], sQ[s])
        pltpu.async_copy(r_hbm.at[ib[s].at[0]], rb[s], sR[s])
        pltpu.async_copy(t_hbm.at[pl.ds(tbase, CHUNK)], tb[s], sT[s])

    def wait_gathers(s):
        pltpu.make_async_copy(q_hbm.at[ib[s].at[1]], qb[s], sQ[s]).wait()
        pltpu.make_async_copy(r_hbm.at[ib[s].at[0]], rb[s], sR[s]).wait()
        pltpu.make_async_copy(
            t_hbm.at[pl.ds(0, CHUNK)], tb[s], sT[s]).wait()

    def issue_idx(g, s):
        row = jnp.minimum(row0 + g, N_CHUNK_ROWS - 1)
        pltpu.async_copy(ei_hbm.at[row], ib[s], sI[s])

    def wait_idx(s):
        pltpu.make_async_copy(ei_hbm.at[0], ib[s], sI[s]).wait()

    def compute(s):
        @plsc.parallel_loop(0, CHUNK, unroll=2)
        def edge_body(e):
            for j in range(N_GROUPS):
                lo = 16 * j
                hi = D_HID + lo
                af = (qb[s][e, pl.ds(lo, LANES)] + rb[s][e, pl.ds(lo, LANES)]
                      + tb[s][e, pl.ds(lo, LANES)])
                av = (qb[s][e, pl.ds(hi, LANES)] + rb[s][e, pl.ds(hi, LANES)]
                      + tb[s][e, pl.ds(hi, LANES)])
                mb[s][e, pl.ds(lo, LANES)] = _gates(af, av)
        pltpu.sync_copy(mb[s], agg.at[ib[s].at[2]], add=True)

    def steady(g, p, q):
        wait_gathers(p)
        wait_idx(q)
        issue_gathers(g + 1, q)
        compute(p)
        issue_idx(g + 2, p)

    pltpu.sync_copy(ei_hbm.at[row0], ib[0])
    issue_gathers(0, 0)
    issue_idx(1, 1)

    def pair_body(k, c):
        steady(2 * k, 0, 1)
        steady(2 * k + 1, 1, 0)
        return c

    lax.fori_loop(0, (N_CHUNKS - 2) // 2, pair_body, 0)
    steady(N_CHUNKS - 2, 0, 1)

    wait_gathers(1)
    compute(1)
    wait_idx(0)

    plsc.subcore_barrier()

    def out_body(k, c):
        off = pl.multiple_of((k * SC_SUBCORES + sid) * STRIPE_ROWS, 8)
        pltpu.sync_copy(agg.at[pl.ds(off, STRIPE_ROWS)],
                        out_hbm.at[cid, pl.ds(off, STRIPE_ROWS)])
        return c

    lax.fori_loop(0, n_stripes, out_body, 0)


def _make_sc_layer():
    return pl.kernel(
        _sc_layer_body,
        out_type=jax.ShapeDtypeStruct((SC_CORES, N_NODES, D_HID), jnp.float32),
        mesh=plsc.VectorSubcoreMesh(
            core_axis_name="c", subcore_axis_name="s",
            num_cores=SC_CORES, num_subcores=SC_SUBCORES),
        scratch_types=(
            [pltpu.VMEM((3, CHUNK), jnp.int32)] * 2
            + [pltpu.VMEM((CHUNK, 2 * D_HID), jnp.float32)] * 6
            + [pltpu.VMEM((CHUNK, D_HID), jnp.float32)] * 2
            + [pltpu.VMEM_SHARED((N_ACC, D_HID), jnp.float32)]
            + [pltpu.SemaphoreType.DMA] * 8
        ),
    )


def kernel(x, edge_index, edge_attr, batch, Wpre, bpre, Wf1, bf1, Ws1, bs1,
           Wf2, bf2, Ws2, bs2, W1, b1, W2, b2, Wout, bout):
    pad_n = EPW_PAD - EDGES_PER_WORKER
    src_w = edge_index[0].reshape(N_WORKERS, EDGES_PER_WORKER)
    dst_w = edge_index[1].reshape(N_WORKERS, EDGES_PER_WORKER)
    src_p = jnp.pad(src_w, ((0, 0), (0, pad_n))).reshape(-1, CHUNK)
    dstg_p = jnp.pad(dst_w, ((0, 0), (0, pad_n))).reshape(-1, CHUNK)
    dsts_p = jnp.pad(dst_w, ((0, 0), (0, pad_n)),
                     constant_values=GARBAGE_ROW).reshape(-1, CHUNK)
    ei3 = jnp.stack([src_p, dstg_p, dsts_p], axis=1)
    ea_p = jnp.pad(edge_attr.reshape(N_WORKERS, EDGES_PER_WORKER, D_EDGE),
                   ((0, 0), (0, pad_n), (0, 0))).reshape(E_PAD, D_EDGE)

    wq1 = jnp.concatenate([Wf1[:D_HID], Ws1[:D_HID]], axis=1)
    wr1 = jnp.concatenate(
        [Wf1[D_HID:2 * D_HID], Ws1[D_HID:2 * D_HID]], axis=1)
    wq2 = jnp.concatenate([Wf2[:D_HID], Ws2[:D_HID]], axis=1)
    wr2 = jnp.concatenate(
        [Wf2[D_HID:2 * D_HID], Ws2[D_HID:2 * D_HID]], axis=1)
    we1 = jnp.concatenate([Wf1[2 * D_HID:], Ws1[2 * D_HID:]], axis=1)
    we2 = jnp.concatenate([Wf2[2 * D_HID:], Ws2[2 * D_HID:]], axis=1)
    be1 = jnp.concatenate([bf1, bs1]).reshape(1, 2 * D_HID)
    be2 = jnp.concatenate([bf2, bs2]).reshape(1, 2 * D_HID)

    h0, q1, r1 = pl.pallas_call(
        _node_proj_body,
        grid=(N_NODE_BLKS,),
        in_specs=[
            _rows(NODE_BLK, D_HID),
            _full((D_HID, D_HID)),
            _full((1, D_HID)),
            _full((D_HID, 2 * D_HID)),
            _full((D_HID, 2 * D_HID)),
        ],
        out_specs=[
            _rows(NODE_BLK, D_HID),
            _rows(NODE_BLK, 2 * D_HID),
            _rows(NODE_BLK, 2 * D_HID),
        ],
        out_shape=[
            jax.ShapeDtypeStruct((N_NODES, D_HID), jnp.float32),
            jax.ShapeDtypeStruct((N_NODES, 2 * D_HID), jnp.float32),
            jax.ShapeDtypeStruct((N_NODES, 2 * D_HID), jnp.float32),
        ],
    )(x, Wpre, bpre.reshape(1, D_HID), wq1, wr1)

    t1, t2 = pl.pallas_call(
        _edge_proj_body,
        grid=(N_EDGE_BLKS,),
        in_specs=[
            _rows(EDGE_BLK, D_EDGE),
            _full((D_EDGE, 2 * D_HID)),
            _full((1, 2 * D_HID)),
            _full((D_EDGE, 2 * D_HID)),
            _full((1, 2 * D_HID)),
        ],
        out_specs=[
            _rows(EDGE_BLK, 2 * D_HID),
            _rows(EDGE_BLK, 2 * D_HID),
        ],
        out_shape=[
            jax.ShapeDtypeStruct((E_PAD, 2 * D_HID), jnp.float32),
            jax.ShapeDtypeStruct((E_PAD, 2 * D_HID), jnp.float32),
        ],
    )(ea_p, we1, be1, we2, be2)

    zeros = jnp.zeros((N_NODES, D_HID), jnp.float32)

    sc_layer = _make_sc_layer()
    agg1 = sc_layer(q1, r1, t1, ei3, zeros)

    h1, q2, r2 = pl.pallas_call(
        _mid_proj_body,
        grid=(N_NODE_BLKS,),
        in_specs=[
            _rows(NODE_BLK, D_HID),
            _rows(NODE_BLK, D_HID),
            _rows(NODE_BLK, D_HID),
            _full((D_HID, 2 * D_HID)),
            _full((D_HID, 2 * D_HID)),
        ],
        out_specs=[
            _rows(NODE_BLK, D_HID),
            _rows(NODE_BLK, 2 * D_HID),
            _rows(NODE_BLK, 2 * D_HID),
        ],
        out_shape=[
            jax.ShapeDtypeStruct((N_NODES, D_HID), jnp.float32),
            jax.ShapeDtypeStruct((N_NODES, 2 * D_HID), jnp.float32),
            jax.ShapeDtypeStruct((N_NODES, 2 * D_HID), jnp.float32),
        ],
    )(h0, agg1[0], agg1[1], wq2, wr2)

    agg2 = sc_layer(q2, r2, t2, ei3, zeros)

    out = pl.pallas_call(
        _pool_head_body,
        grid=(N_NODE_BLKS,),
        in_specs=[
            _rows(NODE_BLK, D_HID),
            _rows(NODE_BLK, D_HID),
            _rows(NODE_BLK, D_HID),
            pl.BlockSpec((1, 1, NODE_BLK), lambda i: (i, 0, 0)),
            _full((D_HID, D_HID)),
            _full((1, D_HID)),
            _full((D_HID, D_HID)),
            _full((1, D_HID)),
            _full((D_HID, 3)),
            _full((1, 3)),
        ],
        out_specs=pl.BlockSpec((N_GRAPHS, 3), lambda i: (0, 0)),
        out_shape=jax.ShapeDtypeStruct((N_GRAPHS, 3), jnp.float32),
        scratch_shapes=[
            pltpu.VMEM((N_GRAPHS, D_HID), jnp.float32),
            pltpu.VMEM((N_GRAPHS, D_HID), jnp.float32),
        ],
    )(h1, agg2[0], agg2[1], batch.reshape(N_NODE_BLKS, 1, NODE_BLK),
      W1, b1.reshape(1, D_HID), W2, b2.reshape(1, D_HID),
      Wout, bout.reshape(1, 3))

    return out


# speedup vs baseline: 2.9507x; 1.1953x over previous
import jax
import jax.numpy as jnp
from jax import lax
from jax.experimental import pallas as pl
from jax.experimental.pallas import tpu as pltpu
from jax.experimental.pallas import tpu_sc as plsc

N_NODES = 10000
N_EDGES = 320000
D_HID = 128
D_HALF = D_HID // 2
D_EDGE = 16
N_GRAPHS = 64

SC_CORES = 2
SC_SUBCORES = 16
LANES = 16
N_GROUPS = D_HALF // LANES
CHUNK = 40
RING = 2
EPW = N_EDGES // SC_SUBCORES
EPW_PAD = 20120
N_CHUNKS = EPW_PAD // CHUNK
N_STEADY = N_CHUNKS - 2
E_PAD = SC_SUBCORES * EPW_PAD
N_CHUNK_ROWS = E_PAD // CHUNK
E_PAD2 = 322560
N_ACC = N_NODES + 8
GARBAGE_ROW = N_NODES
STRIPE_ROWS = 40
N_STRIPES = N_NODES // STRIPE_ROWS

NODE_BLK = 1000
N_NODE_BLKS = N_NODES // NODE_BLK
EDGE_BLK = 2240
N_EDGE_BLKS = E_PAD2 // EDGE_BLK

_LOG1P = (
    2.2132784001038797e-05, 0.9990102089269602, -0.48915578201144777,
    0.28330238362042115, -0.13011793028847676, 0.030102247599677626,
)


def _dot(a, b):
    a16 = a.astype(jnp.bfloat16).astype(jnp.float32)
    b16 = b.astype(jnp.bfloat16).astype(jnp.float32)
    return jnp.dot(a16, b16, preferred_element_type=jnp.float32,
                   precision=lax.Precision.HIGHEST)


def _dot_exact(a, b):
    return jnp.dot(a, b, preferred_element_type=jnp.float32,
                   precision=lax.Precision.HIGHEST)


def _node_proj_body(x_ref, wpre_ref, bpre_ref, wq_ref, wr_ref,
                    h_ref, q_ref, r_ref):
    h = jnp.maximum(_dot(x_ref[:], wpre_ref[:]) + bpre_ref[:], 0.0)
    h_ref[:] = h
    q_ref[:] = _dot(h, wq_ref[0])
    r_ref[:] = _dot(h, wr_ref[0])


def _mid_proj_body(h_ref, a0_ref, a1_ref, wq_ref, wr_ref,
                   hn_ref, q_ref, r_ref):
    agg = jnp.concatenate([a0_ref[:][:, :D_HALF], a1_ref[:][:, :D_HALF]],
                          axis=1)
    h = jnp.maximum(h_ref[:] + agg, 0.0)
    hn_ref[:] = h
    q_ref[:] = _dot(h, wq_ref[0])
    r_ref[:] = _dot(h, wr_ref[0])


def _edge_proj_body(ea_ref, w1_ref, b1_ref, w2_ref, b2_ref, t1_ref, t2_ref):
    ea = ea_ref[:]
    t1_ref[:] = _dot(ea, w1_ref[0]) + b1_ref[0]
    t2_ref[:] = _dot(ea, w2_ref[0]) + b2_ref[0]


def _pool_head_body(h_ref, a0_ref, a1_ref, batch_ref,
                    w1_ref, b1_ref, w2_ref, b2_ref, wo_ref, bo_ref,
                    out_ref, sum_acc, cnt_acc):
    i = pl.program_id(0)

    @pl.when(i == 0)
    def _():
        sum_acc[:] = jnp.zeros_like(sum_acc)
        cnt_acc[:] = jnp.zeros_like(cnt_acc)

    agg = jnp.concatenate([a0_ref[:][:, :D_HALF], a1_ref[:][:, :D_HALF]],
                          axis=1)
    h2 = jnp.maximum(h_ref[:] + agg, 0.0)
    b = batch_ref[0, 0, :]
    gids = lax.broadcasted_iota(jnp.int32, (N_GRAPHS, NODE_BLK), 0)
    onehot_t = (b[None, :] == gids).astype(jnp.float32)
    sum_acc[:] += _dot_exact(onehot_t, h2)
    cnt = jnp.sum(onehot_t, axis=1, keepdims=True)
    cnt_acc[:] += jnp.broadcast_to(cnt, cnt_acc.shape)

    @pl.when(i == N_NODE_BLKS - 1)
    def _():
        pooled = sum_acc[:] / jnp.maximum(cnt_acc[:], 1.0)
        o = jnp.maximum(_dot(pooled, w1_ref[:]) + b1_ref[:], 0.0)
        o = jnp.maximum(_dot(o, w2_ref[:]) + b2_ref[:], 0.0)
        out_ref[:] = _dot(o, wo_ref[:]) + bo_ref[:]


def _full(shape):
    return pl.BlockSpec(shape, lambda i: (0,) * len(shape))


def _rows(blk, width):
    return pl.BlockSpec((blk, width), lambda i: (i, 0))


def _gates(af, av):
    sig = 1.0 / (1.0 + jnp.exp(-af))
    t = jnp.exp(-jnp.abs(av))
    p = jnp.full((LANES,), _LOG1P[5], jnp.float32)
    for c in _LOG1P[4::-1]:
        p = p * t + c
    sp = jnp.maximum(av, 0.0) + p
    return sig * sp


def _sc_layer_body(q_hbm, r_hbm, t_hbm,
                   ei_hbm, zero_hbm, out_hbm,
                   ib0, ib1, vq0, vq1,
                   vr0, vr1, vt0, vt1,
                   vm0, vm1, agg,
                   sQ0, sQ1, sR0, sR1,
                   sT0, sT1, sI0, sI1):
    cid = lax.axis_index("c")
    sid = lax.axis_index("s")
    row0 = sid * N_CHUNKS

    ib = (ib0, ib1)
    vq = (vq0, vq1)
    vr = (vr0, vr1)
    vt = (vt0, vt1)
    vm = (vm0, vm1)
    sQ = (sQ0, sQ1)
    sR = (sR0, sR1)
    sT = (sT0, sT1)
    sI = (sI0, sI1)

    def vmz_body(e, c):
        for s in range(RING):
            for j in range(N_GROUPS):
                vm[s][e, pl.ds(D_HALF + 16 * j, LANES)] = (
                    jnp.zeros((LANES,), jnp.float32))
        return c

    lax.fori_loop(0, CHUNK, vmz_body, 0)

    n_stripes = (N_STRIPES - 1 - sid) // SC_SUBCORES + 1

    def zero_body(k, c):
        off = pl.multiple_of((k * SC_SUBCORES + sid) * STRIPE_ROWS, 8)
        pltpu.sync_copy(zero_hbm.at[pl.ds(off, STRIPE_ROWS)],
                        agg.at[pl.ds(off, STRIPE_ROWS)])
        return c

    lax.fori_loop(0, n_stripes, zero_body, 0)
    plsc.subcore_barrier()

    def issue_gathers(g, s):
        tbase = pl.multiple_of(cid * E_PAD2 + (row0 + g) * CHUNK, 8)
        pltpu.async_copy(q_hbm.at[ib[s].at[1]], vq[s], sQ[s])
        pltpu.async_copy(r_hbm.at[ib[s].at[0]], vr[s], sR[s])
        pltpu.async_copy(t_hbm.at[pl.ds(tbase, CHUNK)], vt[s], sT[s])

    def wait_gathers(s):
        pltpu.make_async_copy(q_hbm.at[ib[s].at[1]], vq[s], sQ[s]).wait()
        pltpu.make_async_copy(r_hbm.at[ib[s].at[0]], vr[s], sR[s]).wait()
        pltpu.make_async_copy(
            t_hbm.at[pl.ds(0, CHUNK)], vt[s], sT[s]).wait()

    def issue_idx(g, s):
        row = jnp.minimum(cid * N_CHUNK_ROWS + row0 + g,
                          cid * N_CHUNK_ROWS + N_CHUNK_ROWS - 1)
        pltpu.async_copy(ei_hbm.at[row], ib[s], sI[s])

    def wait_idx(s):
        pltpu.make_async_copy(ei_hbm.at[0], ib[s], sI[s]).wait()

    def compute(s):
        @plsc.parallel_loop(0, CHUNK, unroll=2)
        def edge_body(e):
            for j in range(N_GROUPS):
                lo = 16 * j
                hi = D_HALF + lo
                af = (vq[s][e, pl.ds(lo, LANES)] + vr[s][e, pl.ds(lo, LANES)]
                      + vt[s][e, pl.ds(lo, LANES)])
                av = (vq[s][e, pl.ds(hi, LANES)] + vr[s][e, pl.ds(hi, LANES)]
                      + vt[s][e, pl.ds(hi, LANES)])
                vm[s][e, pl.ds(lo, LANES)] = _gates(af, av)
        pltpu.sync_copy(vm[s], agg.at[ib[s].at[2]], add=True)

    def steady(g, p, f):
        wait_gathers(p)
        wait_idx(f)
        issue_gathers(g + 1, f)
        compute(p)
        issue_idx(g + 2, p)

    pltpu.sync_copy(ei_hbm.at[cid * N_CHUNK_ROWS + row0], ib[0])
    issue_gathers(0, 0)
    issue_idx(1, 1)

    def pair_body(k, c):
        g = 2 * k
        steady(g, 0, 1)
        steady(g + 1, 1, 0)
        return c

    lax.fori_loop(0, (N_CHUNKS - 1) // 2, pair_body, 0)

    wait_gathers(0)
    compute(0)
    wait_idx(1)

    plsc.subcore_barrier()

    def out_body(k, c):
        off = pl.multiple_of((k * SC_SUBCORES + sid) * STRIPE_ROWS, 8)
        pltpu.sync_copy(agg.at[pl.ds(off, STRIPE_ROWS)],
                        out_hbm.at[cid, pl.ds(off, STRIPE_ROWS)])
        return c

    lax.fori_loop(0, n_stripes, out_body, 0)


def _make_sc_layer():
    return pl.kernel(
        _sc_layer_body,
        out_type=jax.ShapeDtypeStruct((SC_CORES, N_NODES, D_HID),
                                      jnp.float32),
        mesh=plsc.VectorSubcoreMesh(
            core_axis_name="c", subcore_axis_name="s",
            num_cores=SC_CORES, num_subcores=SC_SUBCORES),
        scratch_types=(
            [pltpu.VMEM((3, CHUNK), jnp.int32)] * RING
            + [pltpu.VMEM((CHUNK, 2 * D_HALF), jnp.float32)] * (3 * RING)
            + [pltpu.VMEM((CHUNK, 2 * D_HALF), jnp.float32)] * RING
            + [pltpu.VMEM_SHARED((N_ACC, 2 * D_HALF), jnp.float32)]
            + [pltpu.SemaphoreType.DMA] * (4 * RING)
        ),
    )


def kernel(x, edge_index, edge_attr, batch, Wpre, bpre, Wf1, bf1, Ws1, bs1,
           Wf2, bf2, Ws2, bs2, W1, b1, W2, b2, Wout, bout):
    pad_n = EPW_PAD - EPW
    src_w = edge_index[0].reshape(SC_SUBCORES, EPW)
    dst_w = edge_index[1].reshape(SC_SUBCORES, EPW)
    src_p = jnp.pad(src_w, ((0, 0), (0, pad_n))).reshape(-1, CHUNK)
    dstg_p = jnp.pad(dst_w, ((0, 0), (0, pad_n))).reshape(-1, CHUNK)
    dsts_p = jnp.pad(dst_w, ((0, 0), (0, pad_n)),
                     constant_values=GARBAGE_ROW).reshape(-1, CHUNK)
    ei0 = jnp.stack([src_p, dstg_p, dsts_p], axis=1)
    off = jnp.array([N_NODES, N_NODES, 0], jnp.int32).reshape(1, 3, 1)
    ei4 = jnp.concatenate([ei0, ei0 + off], axis=0)
    ea_p = jnp.pad(edge_attr.reshape(SC_SUBCORES, EPW, D_EDGE),
                   ((0, 0), (0, pad_n), (0, 0))).reshape(E_PAD, D_EDGE)
    ea_p = jnp.pad(ea_p, ((0, E_PAD2 - E_PAD), (0, 0)))

    def _wstack(wf, ws):
        return jnp.stack(
            [jnp.concatenate([wf[:, :D_HALF], ws[:, :D_HALF]], axis=1),
             jnp.concatenate([wf[:, D_HALF:], ws[:, D_HALF:]], axis=1)])

    wq1 = _wstack(Wf1[:D_HID], Ws1[:D_HID])
    wr1 = _wstack(Wf1[D_HID:2 * D_HID], Ws1[D_HID:2 * D_HID])
    wq2 = _wstack(Wf2[:D_HID], Ws2[:D_HID])
    wr2 = _wstack(Wf2[D_HID:2 * D_HID], Ws2[D_HID:2 * D_HID])
    we1 = _wstack(Wf1[2 * D_HID:], Ws1[2 * D_HID:])
    we2 = _wstack(Wf2[2 * D_HID:], Ws2[2 * D_HID:])

    def _bstack(bf, bs):
        return jnp.stack(
            [jnp.concatenate([bf[:D_HALF], bs[:D_HALF]]),
             jnp.concatenate([bf[D_HALF:], bs[D_HALF:]])]).reshape(
                 SC_CORES, 1, 2 * D_HALF)

    be1 = _bstack(bf1, bs1)
    be2 = _bstack(bf2, bs2)

    wspec = pl.BlockSpec((1, D_HID, 2 * D_HALF), lambda c, i: (c, 0, 0))
    wespec = pl.BlockSpec((1, D_EDGE, 2 * D_HALF), lambda c, i: (c, 0, 0))
    bespec = pl.BlockSpec((1, 1, 2 * D_HALF), lambda c, i: (c, 0, 0))
    nrows = pl.BlockSpec((NODE_BLK, D_HID), lambda c, i: (i, 0))
    nhalf = pl.BlockSpec((NODE_BLK, D_HALF), lambda c, i: (i, 0))
    stackrows = pl.BlockSpec((NODE_BLK, 2 * D_HALF),
                             lambda c, i: (c * N_NODE_BLKS + i, 0))
    nodeT = jax.ShapeDtypeStruct((SC_CORES * N_NODES, 2 * D_HALF),
                                 jnp.float32)

    h0, q1, r1 = pl.pallas_call(
        _node_proj_body,
        grid=(SC_CORES, N_NODE_BLKS),
        in_specs=[nrows,
                  pl.BlockSpec((D_HID, D_HID), lambda c, i: (0, 0)),
                  pl.BlockSpec((1, D_HID), lambda c, i: (0, 0)),
                  wspec, wspec],
        out_specs=[nrows, stackrows, stackrows],
        out_shape=[jax.ShapeDtypeStruct((N_NODES, D_HID), jnp.float32),
                   nodeT, nodeT],
    )(x, Wpre, bpre.reshape(1, D_HID), wq1, wr1)

    erows = pl.BlockSpec((EDGE_BLK, D_EDGE), lambda c, i: (i, 0))
    estackrows = pl.BlockSpec((EDGE_BLK, 2 * D_HALF),
                              lambda c, i: (c * N_EDGE_BLKS + i, 0))
    edgeT = jax.ShapeDtypeStruct((SC_CORES * E_PAD2, 2 * D_HALF),
                                 jnp.float32)
    t1, t2 = pl.pallas_call(
        _edge_proj_body,
        grid=(SC_CORES, N_EDGE_BLKS),
        in_specs=[erows, wespec, bespec, wespec, bespec],
        out_specs=[estackrows, estackrows],
        out_shape=[edgeT, edgeT],
    )(ea_p, we1, be1, we2, be2)

    zeros = jnp.zeros((N_NODES, D_HID), jnp.float32)

    sc_layer = _make_sc_layer()
    agg1 = sc_layer(q1, r1, t1, ei4, zeros)

    h1, q2, r2 = pl.pallas_call(
        _mid_proj_body,
        grid=(SC_CORES, N_NODE_BLKS),
        in_specs=[nrows, nrows, nrows, wspec, wspec],
        out_specs=[nrows, stackrows, stackrows],
        out_shape=[jax.ShapeDtypeStruct((N_NODES, D_HID), jnp.float32),
                   nodeT, nodeT],
    )(h0, agg1[0], agg1[1], wq2, wr2)

    agg2 = sc_layer(q2, r2, t2, ei4, zeros)

    out = pl.pallas_call(
        _pool_head_body,
        grid=(N_NODE_BLKS,),
        in_specs=[
            _rows(NODE_BLK, D_HID),
            _rows(NODE_BLK, D_HID),
            _rows(NODE_BLK, D_HID),
            pl.BlockSpec((1, 1, NODE_BLK), lambda i: (i, 0, 0)),
            _full((D_HID, D_HID)),
            _full((1, D_HID)),
            _full((D_HID, D_HID)),
            _full((1, D_HID)),
            _full((D_HID, 3)),
            _full((1, 3)),
        ],
        out_specs=pl.BlockSpec((N_GRAPHS, 3), lambda i: (0, 0)),
        out_shape=jax.ShapeDtypeStruct((N_GRAPHS, 3), jnp.float32),
        scratch_shapes=[
            pltpu.VMEM((N_GRAPHS, D_HID), jnp.float32),
            pltpu.VMEM((N_GRAPHS, D_HID), jnp.float32),
        ],
    )(h1, agg2[0], agg2[1], batch.reshape(N_NODE_BLKS, 1, NODE_BLK),
      W1, b1.reshape(1, D_HID), W2, b2.reshape(1, D_HID),
      Wout, bout.reshape(1, 3))

    return out


# speedup vs baseline: 2.9872x; 1.0124x over previous
import jax
import jax.numpy as jnp
from jax import lax
from jax.experimental import pallas as pl
from jax.experimental.pallas import tpu as pltpu
from jax.experimental.pallas import tpu_sc as plsc

N_NODES = 10000
N_EDGES = 320000
D_HID = 128
D_HALF = D_HID // 2
D_EDGE = 16
N_GRAPHS = 64

SC_CORES = 2
SC_SUBCORES = 16
LANES = 16
N_GROUPS = D_HALF // LANES
CHUNK = 40
RING = 2
EPW = N_EDGES // SC_SUBCORES
EPW_PAD = 20120
N_CHUNKS = EPW_PAD // CHUNK
N_STEADY = N_CHUNKS - 2
E_PAD = SC_SUBCORES * EPW_PAD
N_CHUNK_ROWS = E_PAD // CHUNK
E_PAD2 = 322560
N_ACC = N_NODES + 8
GARBAGE_ROW = N_NODES
STRIPE_ROWS = 40
N_STRIPES = N_NODES // STRIPE_ROWS

NODE_BLK = 1000
N_NODE_BLKS = N_NODES // NODE_BLK
EDGE_BLK = 2240
N_EDGE_BLKS = E_PAD2 // EDGE_BLK

_LOG1P = (
    2.2132784001038797e-05, 0.9990102089269602, -0.48915578201144777,
    0.28330238362042115, -0.13011793028847676, 0.030102247599677626,
)


def _dot(a, b):
    a16 = a.astype(jnp.bfloat16).astype(jnp.float32)
    b16 = b.astype(jnp.bfloat16).astype(jnp.float32)
    return jnp.dot(a16, b16, preferred_element_type=jnp.float32,
                   precision=lax.Precision.HIGHEST)


def _dot_exact(a, b):
    return jnp.dot(a, b, preferred_element_type=jnp.float32,
                   precision=lax.Precision.HIGHEST)


def _node_proj_body(x_ref, wpre_ref, bpre_ref, wq_ref, wr_ref,
                    h_ref, q_ref, r_ref):
    h = jnp.maximum(_dot(x_ref[:], wpre_ref[:]) + bpre_ref[:], 0.0)
    h_ref[:] = h
    q_ref[:] = _dot(h, wq_ref[0])
    r_ref[:] = _dot(h, wr_ref[0])


def _mid_proj_body(h_ref, a0_ref, a1_ref, wq_ref, wr_ref,
                   hn_ref, q_ref, r_ref):
    agg = jnp.concatenate([a0_ref[:][:, :D_HALF], a1_ref[:][:, :D_HALF]],
                          axis=1)
    h = jnp.maximum(h_ref[:] + agg, 0.0)
    hn_ref[:] = h
    q_ref[:] = _dot(h, wq_ref[0])
    r_ref[:] = _dot(h, wr_ref[0])


def _edge_proj_body(ea_ref, w1_ref, b1_ref, w2_ref, b2_ref, t1_ref, t2_ref):
    ea = ea_ref[:]
    t1_ref[:] = _dot(ea, w1_ref[0]) + b1_ref[0]
    t2_ref[:] = _dot(ea, w2_ref[0]) + b2_ref[0]


def _pool_head_body(h_ref, a0_ref, a1_ref, batch_ref,
                    w1_ref, b1_ref, w2_ref, b2_ref, wo_ref, bo_ref,
                    out_ref, sum_acc, cnt_acc):
    i = pl.program_id(0)

    @pl.when(i == 0)
    def _():
        sum_acc[:] = jnp.zeros_like(sum_acc)
        cnt_acc[:] = jnp.zeros_like(cnt_acc)

    agg = jnp.concatenate([a0_ref[:][:, :D_HALF], a1_ref[:][:, :D_HALF]],
                          axis=1)
    h2 = jnp.maximum(h_ref[:] + agg, 0.0)
    b = batch_ref[0, 0, :]
    gids = lax.broadcasted_iota(jnp.int32, (N_GRAPHS, NODE_BLK), 0)
    onehot_t = (b[None, :] == gids).astype(jnp.float32)
    sum_acc[:] += _dot_exact(onehot_t, h2)
    cnt = jnp.sum(onehot_t, axis=1, keepdims=True)
    cnt_acc[:] += jnp.broadcast_to(cnt, cnt_acc.shape)

    @pl.when(i == N_NODE_BLKS - 1)
    def _():
        pooled = sum_acc[:] / jnp.maximum(cnt_acc[:], 1.0)
        o = jnp.maximum(_dot(pooled, w1_ref[:]) + b1_ref[:], 0.0)
        o = jnp.maximum(_dot(o, w2_ref[:]) + b2_ref[:], 0.0)
        out_ref[:] = _dot(o, wo_ref[:]) + bo_ref[:]


def _full(shape):
    return pl.BlockSpec(shape, lambda i: (0,) * len(shape))


def _rows(blk, width):
    return pl.BlockSpec((blk, width), lambda i: (i, 0))


def _gates(af, av):
    sig = 1.0 / (1.0 + jnp.exp(-af))
    t = jnp.exp(-jnp.abs(av))
    p = jnp.full((LANES,), _LOG1P[5], jnp.float32)
    for c in _LOG1P[4::-1]:
        p = p * t + c
    sp = jnp.maximum(av, 0.0) + p
    return sig * sp


def _sc_layer_body(q_hbm, r_hbm, t_hbm,
                   ei_hbm, zero_hbm, out_hbm,
                   ib0, ib1, vq0, vq1,
                   vr0, vr1, vt0, vt1,
                   vm0, vm1, agg,
                   sQ0, sQ1, sR0, sR1,
                   sT0, sT1, sI0, sI1):
    cid = lax.axis_index("c")
    sid = lax.axis_index("s")
    row0 = sid * N_CHUNKS

    ib = (ib0, ib1)
    vq = (vq0, vq1)
    vr = (vr0, vr1)
    vt = (vt0, vt1)
    vm = (vm0, vm1)
    sQ = (sQ0, sQ1)
    sR = (sR0, sR1)
    sT = (sT0, sT1)
    sI = (sI0, sI1)

    def vmz_body(e, c):
        for s in range(RING):
            for j in range(N_GROUPS):
                vm[s][e, pl.ds(D_HALF + 16 * j, LANES)] = (
                    jnp.zeros((LANES,), jnp.float32))
        return c

    lax.fori_loop(0, CHUNK, vmz_body, 0)

    n_stripes = (N_STRIPES - 1 - sid) // SC_SUBCORES + 1

    def zero_body(k, c):
        off = pl.multiple_of((k * SC_SUBCORES + sid) * STRIPE_ROWS, 8)
        pltpu.sync_copy(zero_hbm.at[pl.ds(off, STRIPE_ROWS)],
                        agg.at[pl.ds(off, STRIPE_ROWS)])
        return c

    lax.fori_loop(0, n_stripes, zero_body, 0)
    plsc.subcore_barrier()

    def issue_gathers(g, s):
        tbase = pl.multiple_of(cid * E_PAD2 + (row0 + g) * CHUNK, 8)
        pltpu.async_copy(q_hbm.at[ib[s].at[1]], vq[s], sQ[s])
        pltpu.async_copy(r_hbm.at[ib[s].at[0]], vr[s], sR[s])
        pltpu.async_copy(t_hbm.at[pl.ds(tbase, CHUNK)], vt[s], sT[s])

    def wait_gathers(s):
        pltpu.make_async_copy(q_hbm.at[ib[s].at[1]], vq[s], sQ[s]).wait()
        pltpu.make_async_copy(r_hbm.at[ib[s].at[0]], vr[s], sR[s]).wait()
        pltpu.make_async_copy(
            t_hbm.at[pl.ds(0, CHUNK)], vt[s], sT[s]).wait()

    def issue_idx(g, s):
        row = jnp.minimum(cid * N_CHUNK_ROWS + row0 + g,
                          cid * N_CHUNK_ROWS + N_CHUNK_ROWS - 1)
        pltpu.async_copy(ei_hbm.at[row], ib[s], sI[s])

    def wait_idx(s):
        pltpu.make_async_copy(ei_hbm.at[0], ib[s], sI[s]).wait()

    def compute(s):
        @plsc.parallel_loop(0, CHUNK, unroll=4)
        def edge_body(e):
            for j in range(N_GROUPS):
                lo = 16 * j
                hi = D_HALF + lo
                af = (vq[s][e, pl.ds(lo, LANES)] + vr[s][e, pl.ds(lo, LANES)]
                      + vt[s][e, pl.ds(lo, LANES)])
                av = (vq[s][e, pl.ds(hi, LANES)] + vr[s][e, pl.ds(hi, LANES)]
                      + vt[s][e, pl.ds(hi, LANES)])
                vm[s][e, pl.ds(lo, LANES)] = _gates(af, av)
        pltpu.sync_copy(vm[s], agg.at[ib[s].at[2]], add=True)

    def steady(g, p, f):
        wait_gathers(p)
        wait_idx(f)
        issue_gathers(g + 1, f)
        compute(p)
        issue_idx(g + 2, p)

    pltpu.sync_copy(ei_hbm.at[cid * N_CHUNK_ROWS + row0], ib[0])
    issue_gathers(0, 0)
    issue_idx(1, 1)

    def pair_body(k, c):
        g = 2 * k
        steady(g, 0, 1)
        steady(g + 1, 1, 0)
        return c

    lax.fori_loop(0, (N_CHUNKS - 1) // 2, pair_body, 0)

    wait_gathers(0)
    compute(0)
    wait_idx(1)

    plsc.subcore_barrier()

    def out_body(k, c):
        off = pl.multiple_of((k * SC_SUBCORES + sid) * STRIPE_ROWS, 8)
        pltpu.sync_copy(agg.at[pl.ds(off, STRIPE_ROWS)],
                        out_hbm.at[cid, pl.ds(off, STRIPE_ROWS)])
        return c

    lax.fori_loop(0, n_stripes, out_body, 0)


def _make_sc_layer():
    return pl.kernel(
        _sc_layer_body,
        out_type=jax.ShapeDtypeStruct((SC_CORES, N_NODES, D_HID),
                                      jnp.float32),
        mesh=plsc.VectorSubcoreMesh(
            core_axis_name="c", subcore_axis_name="s",
            num_cores=SC_CORES, num_subcores=SC_SUBCORES),
        scratch_types=(
            [pltpu.VMEM((3, CHUNK), jnp.int32)] * RING
            + [pltpu.VMEM((CHUNK, 2 * D_HALF), jnp.float32)] * (3 * RING)
            + [pltpu.VMEM((CHUNK, 2 * D_HALF), jnp.float32)] * RING
            + [pltpu.VMEM_SHARED((N_ACC, 2 * D_HALF), jnp.float32)]
            + [pltpu.SemaphoreType.DMA] * (4 * RING)
        ),
    )


def kernel(x, edge_index, edge_attr, batch, Wpre, bpre, Wf1, bf1, Ws1, bs1,
           Wf2, bf2, Ws2, bs2, W1, b1, W2, b2, Wout, bout):
    pad_n = EPW_PAD - EPW
    src_w = edge_index[0].reshape(SC_SUBCORES, EPW)
    dst_w = edge_index[1].reshape(SC_SUBCORES, EPW)
    src_p = jnp.pad(src_w, ((0, 0), (0, pad_n))).reshape(-1, CHUNK)
    dstg_p = jnp.pad(dst_w, ((0, 0), (0, pad_n))).reshape(-1, CHUNK)
    dsts_p = jnp.pad(dst_w, ((0, 0), (0, pad_n)),
                     constant_values=GARBAGE_ROW).reshape(-1, CHUNK)
    ei0 = jnp.stack([src_p, dstg_p, dsts_p], axis=1)
    off = jnp.array([N_NODES, N_NODES, 0], jnp.int32).reshape(1, 3, 1)
    ei4 = jnp.concatenate([ei0, ei0 + off], axis=0)
    ea_p = jnp.pad(edge_attr.reshape(SC_SUBCORES, EPW, D_EDGE),
                   ((0, 0), (0, pad_n), (0, 0))).reshape(E_PAD, D_EDGE)
    ea_p = jnp.pad(ea_p, ((0, E_PAD2 - E_PAD), (0, 0)))

    def _wstack(wf, ws):
        return jnp.stack(
            [jnp.concatenate([wf[:, :D_HALF], ws[:, :D_HALF]], axis=1),
             jnp.concatenate([wf[:, D_HALF:], ws[:, D_HALF:]], axis=1)])

    wq1 = _wstack(Wf1[:D_HID], Ws1[:D_HID])
    wr1 = _wstack(Wf1[D_HID:2 * D_HID], Ws1[D_HID:2 * D_HID])
    wq2 = _wstack(Wf2[:D_HID], Ws2[:D_HID])
    wr2 = _wstack(Wf2[D_HID:2 * D_HID], Ws2[D_HID:2 * D_HID])
    we1 = _wstack(Wf1[2 * D_HID:], Ws1[2 * D_HID:])
    we2 = _wstack(Wf2[2 * D_HID:], Ws2[2 * D_HID:])

    def _bstack(bf, bs):
        return jnp.stack(
            [jnp.concatenate([bf[:D_HALF], bs[:D_HALF]]),
             jnp.concatenate([bf[D_HALF:], bs[D_HALF:]])]).reshape(
                 SC_CORES, 1, 2 * D_HALF)

    be1 = _bstack(bf1, bs1)
    be2 = _bstack(bf2, bs2)

    wspec = pl.BlockSpec((1, D_HID, 2 * D_HALF), lambda c, i: (c, 0, 0))
    wespec = pl.BlockSpec((1, D_EDGE, 2 * D_HALF), lambda c, i: (c, 0, 0))
    bespec = pl.BlockSpec((1, 1, 2 * D_HALF), lambda c, i: (c, 0, 0))
    nrows = pl.BlockSpec((NODE_BLK, D_HID), lambda c, i: (i, 0))
    nhalf = pl.BlockSpec((NODE_BLK, D_HALF), lambda c, i: (i, 0))
    stackrows = pl.BlockSpec((NODE_BLK, 2 * D_HALF),
                             lambda c, i: (c * N_NODE_BLKS + i, 0))
    nodeT = jax.ShapeDtypeStruct((SC_CORES * N_NODES, 2 * D_HALF),
                                 jnp.float32)

    h0, q1, r1 = pl.pallas_call(
        _node_proj_body,
        grid=(SC_CORES, N_NODE_BLKS),
        in_specs=[nrows,
                  pl.BlockSpec((D_HID, D_HID), lambda c, i: (0, 0)),
                  pl.BlockSpec((1, D_HID), lambda c, i: (0, 0)),
                  wspec, wspec],
        out_specs=[nrows, stackrows, stackrows],
        out_shape=[jax.ShapeDtypeStruct((N_NODES, D_HID), jnp.float32),
                   nodeT, nodeT],
    )(x, Wpre, bpre.reshape(1, D_HID), wq1, wr1)

    erows = pl.BlockSpec((EDGE_BLK, D_EDGE), lambda c, i: (i, 0))
    estackrows = pl.BlockSpec((EDGE_BLK, 2 * D_HALF),
                              lambda c, i: (c * N_EDGE_BLKS + i, 0))
    edgeT = jax.ShapeDtypeStruct((SC_CORES * E_PAD2, 2 * D_HALF),
                                 jnp.float32)
    t1, t2 = pl.pallas_call(
        _edge_proj_body,
        grid=(SC_CORES, N_EDGE_BLKS),
        in_specs=[erows, wespec, bespec, wespec, bespec],
        out_specs=[estackrows, estackrows],
        out_shape=[edgeT, edgeT],
    )(ea_p, we1, be1, we2, be2)

    zeros = jnp.zeros((N_NODES, D_HID), jnp.float32)

    sc_layer = _make_sc_layer()
    agg1 = sc_layer(q1, r1, t1, ei4, zeros)

    h1, q2, r2 = pl.pallas_call(
        _mid_proj_body,
        grid=(SC_CORES, N_NODE_BLKS),
        in_specs=[nrows, nrows, nrows, wspec, wspec],
        out_specs=[nrows, stackrows, stackrows],
        out_shape=[jax.ShapeDtypeStruct((N_NODES, D_HID), jnp.float32),
                   nodeT, nodeT],
    )(h0, agg1[0], agg1[1], wq2, wr2)

    agg2 = sc_layer(q2, r2, t2, ei4, zeros)

    out = pl.pallas_call(
        _pool_head_body,
        grid=(N_NODE_BLKS,),
        in_specs=[
            _rows(NODE_BLK, D_HID),
            _rows(NODE_BLK, D_HID),
            _rows(NODE_BLK, D_HID),
            pl.BlockSpec((1, 1, NODE_BLK), lambda i: (i, 0, 0)),
            _full((D_HID, D_HID)),
            _full((1, D_HID)),
            _full((D_HID, D_HID)),
            _full((1, D_HID)),
            _full((D_HID, 3)),
            _full((1, 3)),
        ],
        out_specs=pl.BlockSpec((N_GRAPHS, 3), lambda i: (0, 0)),
        out_shape=jax.ShapeDtypeStruct((N_GRAPHS, 3), jnp.float32),
        scratch_shapes=[
            pltpu.VMEM((N_GRAPHS, D_HID), jnp.float32),
            pltpu.VMEM((N_GRAPHS, D_HID), jnp.float32),
        ],
    )(h1, agg2[0], agg2[1], batch.reshape(N_NODE_BLKS, 1, NODE_BLK),
      W1, b1.reshape(1, D_HID), W2, b2.reshape(1, D_HID),
      Wout, bout.reshape(1, 3))

    return out
